# baseline (speedup 1.0000x reference)
import functools

import jax
import jax.numpy as jnp
import numpy as np
from jax import lax
from jax.experimental import pallas as pl
from jax.experimental.pallas import tpu as pltpu
from jax.experimental.pallas import tpu_sc as plsc

_SPARSITY_RATIO = 0.9
_THRESHOLD_PERCENTILE = 0.95

_LANES = 16
_CHUNK = 16384
_NBINS = 65536


def _gat(x, lane_v):
    return lax.gather(
        x,
        lane_v[:, None],
        lax.GatherDimensionNumbers(
            offset_dims=(), collapsed_slice_dims=(0,), start_index_map=(0,)
        ),
        (1,),
        mode=lax.GatherScatterMode.PROMISE_IN_BOUNDS,
    )


def _splat(v):
    return jnp.full((_LANES,), v, jnp.int32)


def _vshr_l(x, k):
    return lax.shift_right_logical(x, jnp.full((_LANES,), k, jnp.int32))


def _sortable_i32(xv):
    b = lax.bitcast_convert_type(xv, jnp.int32)
    neg = lax.shift_right_arithmetic(b, jnp.full((_LANES,), 31, jnp.int32))
    return b ^ (neg | jnp.int32(-2147483648))


def _scan_hist_pair(sbuf, m_v, base0):
    def body(v, carry):
        pv, bstar, pincl, hsel = carry
        ha = sbuf[pl.ds(v * _LANES, _LANES)]
        hb = sbuf[pl.ds(8192 + v * _LANES, _LANES)]
        h = ha + hb
        cs = plsc.cumsum(h)
        pex = pv + cs - h
        cond = pex <= m_v
        pcnt = plsc.all_reduce_population_count(cond)
        found = pcnt > 0
        lane = jnp.maximum(pcnt - 1, 0)
        csel = _gat(cs, lane)
        hcur = _gat(h, lane)
        base_v = base0 + _splat(v * _LANES)
        bstar = jnp.where(found, base_v + pcnt - 1, bstar)
        pincl = jnp.where(found, pv + csel, pincl)
        hsel = jnp.where(found, hcur, hsel)
        pv = pv + _gat(cs, _splat(_LANES - 1))
        return pv, bstar, pincl, hsel

    init = (_splat(0), _splat(-1), _splat(0), _splat(0))
    return lax.fori_loop(0, 8192 // _LANES, body, init)


def _zero_hist(hist):
    zeros = jnp.zeros((_LANES,), jnp.int32)

    def zbody(i, _):
        for u in range(8):
            hist[pl.ds((i * 8 + u) * _LANES, _LANES)] = zeros
        return 0

    lax.fori_loop(0, _NBINS // _LANES // 8, zbody, 0)


_SH_ROUND = 16384


def _merge_scan(shared, sbuf, hist, s, s0, m_v):
    pv = _splat(0)
    bstar = _splat(0)
    pincl = _splat(0)
    hsel = _splat(0)
    for r in range(_NBINS // _SH_ROUND):
        pltpu.sync_copy(hist.at[pl.ds(r * _SH_ROUND, _SH_ROUND)],
                        shared.at[s])
        plsc.subcore_barrier()
        for j in range(_SH_ROUND // 8192):
            pltpu.sync_copy(shared.at[s0, pl.ds(j * 8192, 8192)],
                            sbuf.at[pl.ds(0, 8192)])
            pltpu.sync_copy(shared.at[s0 + 1, pl.ds(j * 8192, 8192)],
                            sbuf.at[pl.ds(8192, 8192)])
            base = r * _SH_ROUND + j * 8192
            cpv, cb, cp, ch = _scan_hist_pair(sbuf, m_v - pv, _splat(base))
            found = cb >= 0
            bstar = jnp.where(found, cb, bstar)
            pincl = jnp.where(found, pv + cp, pincl)
            hsel = jnp.where(found, ch, hsel)
            pv = pv + cpv
        plsc.subcore_barrier()
    return bstar, pincl, hsel


def _sc_select_body(x_hbm, res_hbm, buf, sbuf, hist, resv, shared,
                    *, n_half, k_rank, n_head):
    c = lax.axis_index("c")
    s = lax.axis_index("s")
    head = c * 8 + s // 2
    rowhalf = s % 2
    off0 = rowhalf * n_half
    nchunks = n_half // _CHUNK
    nvec = _CHUNK // _LANES // 8
    ones = jnp.ones((_LANES,), jnp.int32)
    m1 = n_head - k_rank

    def hist_pass(mask_fn):
        def chunk_body(k, _):
            pltpu.sync_copy(
                x_hbm.at[head, pl.ds(off0 + k * _CHUNK, _CHUNK)], buf)

            def vbody(v, _):
                for u in range(8):
                    xv = buf[pl.ds((v * 8 + u) * _LANES, _LANES)]
                    uu = _sortable_i32(xv)
                    mask_fn(uu)
                return 0

            lax.fori_loop(0, nvec, vbody, 0)
            return 0

        lax.fori_loop(0, nchunks, chunk_body, 0)

    _zero_hist(hist)

    def add_top16(uu):
        idx = _vshr_l(uu, 16)
        plsc.addupdate_scatter(hist, [idx], ones)

    hist_pass(add_top16)

    s0 = s - rowhalf
    m1_v = _splat(m1)
    bstar, pincl, hsel = _merge_scan(shared, sbuf, hist, s, s0, m1_v)

    _zero_hist(hist)

    def add_low16(uu):
        top = _vshr_l(uu, 16)
        idx = uu & jnp.int32(0xFFFF)
        plsc.addupdate_scatter(hist, [idx], ones, mask=top == bstar)

    hist_pass(add_low16)

    m2_v = hsel - (pincl - m1_v)
    bstar2, pincl2, hsel2 = _merge_scan(shared, sbuf, hist, s, s0, m2_v)

    u_v = jnp.left_shift(bstar, 16) | bstar2
    count_v = (_splat(n_head) - pincl) + hsel - (pincl2 - hsel2)

    iot = lax.iota(jnp.int32, _LANES)
    resv[...] = jnp.where(iot == 0, u_v, count_v)
    w = c * 16 + s
    pltpu.sync_copy(resv, res_hbm.at[w])


def _mask_kernel(thr_ref, x_ref, mask_ref):
    h = pl.program_id(0)
    mask_ref[0] = x_ref[0] >= thr_ref[h]


def kernel(batch_size, num_heads, seq_len, attention_scores):
    B, H, Sq, Skv = attention_scores.shape
    n = Sq * Skv
    BH = B * H
    x = attention_scores.reshape(BH, Sq, Skv)
    x_flat = attention_scores.reshape(BH, n)

    loc = np.float32(_THRESHOLD_PERCENTILE) * np.float32(n - 1)
    idx_hi = int(np.ceil(np.float64(loc)))
    k_rank = max(1, n - idx_hi)

    mesh = plsc.VectorSubcoreMesh(core_axis_name="c", subcore_axis_name="s")
    sc_select = functools.partial(
        pl.kernel,
        out_type=jax.ShapeDtypeStruct((32, _LANES), jnp.int32),
        mesh=mesh,
        compiler_params=pltpu.CompilerParams(needs_layout_passes=False),
        scratch_types=[
            pltpu.VMEM((_CHUNK,), jnp.float32),
            pltpu.VMEM((2 * 8192,), jnp.int32),
            pltpu.VMEM((_NBINS,), jnp.int32),
            pltpu.VMEM((_LANES,), jnp.int32),
            pltpu.VMEM_SHARED((16, _SH_ROUND), jnp.int32),
        ],
    )(functools.partial(
        _sc_select_body, n_half=n // 2, k_rank=k_rank, n_head=n))

    res = sc_select(x_flat)

    rows = (jnp.arange(BH, dtype=jnp.int32) // 8) * 16 + (
        jnp.arange(BH, dtype=jnp.int32) % 8) * 2
    u_sort = res[rows, 0]
    counts = res[rows, 1]
    bits = jnp.where(
        u_sort < 0, u_sort ^ jnp.int32(-2147483648), ~u_sort
    )
    thr = lax.bitcast_convert_type(bits, jnp.float32)

    row_blk = 256
    mask3 = pl.pallas_call(
        _mask_kernel,
        grid=(BH, Sq // row_blk),
        in_specs=[
            pl.BlockSpec((BH,), lambda i, j: (0,), memory_space=pltpu.SMEM),
            pl.BlockSpec((1, row_blk, Skv), lambda i, j: (i, j, 0)),
        ],
        out_specs=pl.BlockSpec((1, row_blk, Skv), lambda i, j: (i, j, 0)),
        out_shape=jax.ShapeDtypeStruct((BH, Sq, Skv), jnp.bool_),
        compiler_params=pltpu.CompilerParams(
            dimension_semantics=("arbitrary", "arbitrary"),
            vmem_limit_bytes=60 * 1024 * 1024,
        ),
    )(thr, x)

    mask = mask3.reshape(B, H, Sq, Skv)
    density = jnp.sum(counts).astype(jnp.float32) / np.float32(BH * n)

    k = max(1, int(Skv * (1.0 - _SPARSITY_RATIO)))

    def topk_branch():
        _, topk_idx = jax.lax.top_k(attention_scores, k)
        bidx = jnp.arange(B)[:, None, None, None]
        hidx = jnp.arange(H)[None, :, None, None]
        qidx = jnp.arange(Sq)[None, None, :, None]
        topk_mask = jnp.zeros((B, H, Sq, Skv), dtype=bool)
        return topk_mask.at[bidx, hidx, qidx, topk_idx].set(True)

    return jax.lax.cond(
        density > np.float32(1.0 - _SPARSITY_RATIO),
        topk_branch,
        lambda: mask,
    )


# speedup vs baseline: 2.6613x; 2.6613x over previous
import functools

import jax
import jax.numpy as jnp
import numpy as np
from jax import lax
from jax.experimental import pallas as pl
from jax.experimental.pallas import tpu as pltpu
from jax.experimental.pallas import tpu_sc as plsc

_SPARSITY_RATIO = 0.9
_THRESHOLD_PERCENTILE = 0.95

_LANES = 16
_CHUNK = 16384
_NBINS = 65536


def _gat(x, lane_v):
    return lax.gather(
        x,
        lane_v[:, None],
        lax.GatherDimensionNumbers(
            offset_dims=(), collapsed_slice_dims=(0,), start_index_map=(0,)
        ),
        (1,),
        mode=lax.GatherScatterMode.PROMISE_IN_BOUNDS,
    )


def _splat(v):
    return jnp.full((_LANES,), v, jnp.int32)


def _vshr_l(x, k):
    return lax.shift_right_logical(x, jnp.full((_LANES,), k, jnp.int32))


def _sortable_i32(xv):
    b = lax.bitcast_convert_type(xv, jnp.int32)
    neg = lax.shift_right_arithmetic(b, jnp.full((_LANES,), 31, jnp.int32))
    return b ^ (neg | jnp.int32(-2147483648))


def _scan_hist_pair(sbuf, m_v, base0):
    def body(v, carry):
        pv, bstar, pincl, hsel = carry
        ha = sbuf[pl.ds(v * _LANES, _LANES)]
        hb = sbuf[pl.ds(8192 + v * _LANES, _LANES)]
        h = ha + hb
        cs = plsc.cumsum(h)
        pex = pv + cs - h
        cond = pex <= m_v
        pcnt = plsc.all_reduce_population_count(cond)
        found = pcnt > 0
        lane = jnp.maximum(pcnt - 1, 0)
        csel = _gat(cs, lane)
        hcur = _gat(h, lane)
        base_v = base0 + _splat(v * _LANES)
        bstar = jnp.where(found, base_v + pcnt - 1, bstar)
        pincl = jnp.where(found, pv + csel, pincl)
        hsel = jnp.where(found, hcur, hsel)
        pv = pv + _gat(cs, _splat(_LANES - 1))
        return pv, bstar, pincl, hsel

    init = (_splat(0), _splat(-1), _splat(0), _splat(0))
    return lax.fori_loop(0, 8192 // _LANES, body, init)


def _zero_hist(hist):
    zeros = jnp.zeros((_LANES,), jnp.int32)

    def zbody(i, _):
        for u in range(8):
            hist[pl.ds((i * 8 + u) * _LANES, _LANES)] = zeros
        return 0

    lax.fori_loop(0, _NBINS // _LANES // 8, zbody, 0)


_SH_ROUND = 16384


def _merge_scan(shared, sbuf, hist, s, s0, m_v):
    pv = _splat(0)
    bstar = _splat(0)
    pincl = _splat(0)
    hsel = _splat(0)
    for r in range(_NBINS // _SH_ROUND):
        pltpu.sync_copy(hist.at[pl.ds(r * _SH_ROUND, _SH_ROUND)],
                        shared.at[s])
        plsc.subcore_barrier()
        for j in range(_SH_ROUND // 8192):
            pltpu.sync_copy(shared.at[s0, pl.ds(j * 8192, 8192)],
                            sbuf.at[pl.ds(0, 8192)])
            pltpu.sync_copy(shared.at[s0 + 1, pl.ds(j * 8192, 8192)],
                            sbuf.at[pl.ds(8192, 8192)])
            base = r * _SH_ROUND + j * 8192
            cpv, cb, cp, ch = _scan_hist_pair(sbuf, m_v - pv, _splat(base))
            found = cb >= 0
            bstar = jnp.where(found, cb, bstar)
            pincl = jnp.where(found, pv + cp, pincl)
            hsel = jnp.where(found, ch, hsel)
            pv = pv + cpv
        plsc.subcore_barrier()
    return bstar, pincl, hsel


def _sc_select_body(x_hbm, res_hbm, buf, sbuf, hist, resv, shared,
                    *, n_half, k_rank, n_head):
    c = lax.axis_index("c")
    s = lax.axis_index("s")
    head = c * 8 + s // 2
    rowhalf = s % 2
    off0 = rowhalf * n_half
    nchunks = n_half // _CHUNK
    nvec = _CHUNK // _LANES // 8
    ones = jnp.ones((_LANES,), jnp.int32)
    m1 = n_head - k_rank

    def hist_pass(mask_fn):
        def chunk_body(k, _):
            pltpu.sync_copy(
                x_hbm.at[head, pl.ds(off0 + k * _CHUNK, _CHUNK)], buf)

            @plsc.parallel_loop(0, _CHUNK // _LANES, step=1, unroll=8)
            def _(v):
                xv = buf[pl.ds(v * _LANES, _LANES)]
                uu = _sortable_i32(xv)
                mask_fn(uu)

            return 0

        lax.fori_loop(0, nchunks, chunk_body, 0)

    _zero_hist(hist)

    def add_top16(uu):
        idx = _vshr_l(uu, 16)
        plsc.addupdate_scatter(hist, [idx], ones)

    hist_pass(add_top16)

    s0 = s - rowhalf
    m1_v = _splat(m1)
    bstar, pincl, hsel = _merge_scan(shared, sbuf, hist, s, s0, m1_v)

    _zero_hist(hist)

    def add_low16(uu):
        top = _vshr_l(uu, 16)
        idx = uu & jnp.int32(0xFFFF)
        plsc.addupdate_scatter(hist, [idx], ones, mask=top == bstar)

    hist_pass(add_low16)

    m2_v = hsel - (pincl - m1_v)
    bstar2, pincl2, hsel2 = _merge_scan(shared, sbuf, hist, s, s0, m2_v)

    u_v = jnp.left_shift(bstar, 16) | bstar2
    count_v = (_splat(n_head) - pincl) + hsel - (pincl2 - hsel2)

    iot = lax.iota(jnp.int32, _LANES)
    resv[...] = jnp.where(iot == 0, u_v, count_v)
    w = c * 16 + s
    pltpu.sync_copy(resv, res_hbm.at[w])


def _mask_kernel(thr_ref, x_ref, mask_ref):
    h = pl.program_id(0)
    mask_ref[0] = x_ref[0] >= thr_ref[h]


def kernel(batch_size, num_heads, seq_len, attention_scores):
    B, H, Sq, Skv = attention_scores.shape
    n = Sq * Skv
    BH = B * H
    x = attention_scores.reshape(BH, Sq, Skv)
    x_flat = attention_scores.reshape(BH, n)

    loc = np.float32(_THRESHOLD_PERCENTILE) * np.float32(n - 1)
    idx_hi = int(np.ceil(np.float64(loc)))
    k_rank = max(1, n - idx_hi)

    mesh = plsc.VectorSubcoreMesh(core_axis_name="c", subcore_axis_name="s")
    sc_select = functools.partial(
        pl.kernel,
        out_type=jax.ShapeDtypeStruct((32, _LANES), jnp.int32),
        mesh=mesh,
        compiler_params=pltpu.CompilerParams(needs_layout_passes=False),
        scratch_types=[
            pltpu.VMEM((_CHUNK,), jnp.float32),
            pltpu.VMEM((2 * 8192,), jnp.int32),
            pltpu.VMEM((_NBINS,), jnp.int32),
            pltpu.VMEM((_LANES,), jnp.int32),
            pltpu.VMEM_SHARED((16, _SH_ROUND), jnp.int32),
        ],
    )(functools.partial(
        _sc_select_body, n_half=n // 2, k_rank=k_rank, n_head=n))

    res = sc_select(x_flat)

    rows = (jnp.arange(BH, dtype=jnp.int32) // 8) * 16 + (
        jnp.arange(BH, dtype=jnp.int32) % 8) * 2
    u_sort = res[rows, 0]
    counts = res[rows, 1]
    bits = jnp.where(
        u_sort < 0, u_sort ^ jnp.int32(-2147483648), ~u_sort
    )
    thr = lax.bitcast_convert_type(bits, jnp.float32)

    row_blk = 256
    mask3 = pl.pallas_call(
        _mask_kernel,
        grid=(BH, Sq // row_blk),
        in_specs=[
            pl.BlockSpec((BH,), lambda i, j: (0,), memory_space=pltpu.SMEM),
            pl.BlockSpec((1, row_blk, Skv), lambda i, j: (i, j, 0)),
        ],
        out_specs=pl.BlockSpec((1, row_blk, Skv), lambda i, j: (i, j, 0)),
        out_shape=jax.ShapeDtypeStruct((BH, Sq, Skv), jnp.bool_),
        compiler_params=pltpu.CompilerParams(
            dimension_semantics=("arbitrary", "arbitrary"),
            vmem_limit_bytes=60 * 1024 * 1024,
        ),
    )(thr, x)

    mask = mask3.reshape(B, H, Sq, Skv)
    density = jnp.sum(counts).astype(jnp.float32) / np.float32(BH * n)

    k = max(1, int(Skv * (1.0 - _SPARSITY_RATIO)))

    def topk_branch():
        _, topk_idx = jax.lax.top_k(attention_scores, k)
        bidx = jnp.arange(B)[:, None, None, None]
        hidx = jnp.arange(H)[None, :, None, None]
        qidx = jnp.arange(Sq)[None, None, :, None]
        topk_mask = jnp.zeros((B, H, Sq, Skv), dtype=bool)
        return topk_mask.at[bidx, hidx, qidx, topk_idx].set(True)

    return jax.lax.cond(
        density > np.float32(1.0 - _SPARSITY_RATIO),
        topk_branch,
        lambda: mask,
    )


# speedup vs baseline: 3.1544x; 1.1853x over previous
import functools

import jax
import jax.numpy as jnp
import numpy as np
from jax import lax
from jax.experimental import pallas as pl
from jax.experimental.pallas import tpu as pltpu
from jax.experimental.pallas import tpu_sc as plsc

_SPARSITY_RATIO = 0.9
_THRESHOLD_PERCENTILE = 0.95

_LANES = 16
_CHUNK = 16384
_NBINS = 65536


def _gat(x, lane_v):
    return lax.gather(
        x,
        lane_v[:, None],
        lax.GatherDimensionNumbers(
            offset_dims=(), collapsed_slice_dims=(0,), start_index_map=(0,)
        ),
        (1,),
        mode=lax.GatherScatterMode.PROMISE_IN_BOUNDS,
    )


def _splat(v):
    return jnp.full((_LANES,), v, jnp.int32)


def _vshr_l(x, k):
    return lax.shift_right_logical(x, jnp.full((_LANES,), k, jnp.int32))


def _sortable_i32(xv):
    b = lax.bitcast_convert_type(xv, jnp.int32)
    neg = lax.shift_right_arithmetic(b, jnp.full((_LANES,), 31, jnp.int32))
    return b ^ (neg | jnp.int32(-2147483648))


def _scan_hist_pair(sbuf, m_v, base0):
    def body(v, carry):
        pv, bstar, pincl, hsel = carry
        ha = sbuf[pl.ds(v * _LANES, _LANES)]
        hb = sbuf[pl.ds(8192 + v * _LANES, _LANES)]
        h = ha + hb
        cs = plsc.cumsum(h)
        pex = pv + cs - h
        cond = pex <= m_v
        pcnt = plsc.all_reduce_population_count(cond)
        found = pcnt > 0
        lane = jnp.maximum(pcnt - 1, 0)
        csel = _gat(cs, lane)
        hcur = _gat(h, lane)
        base_v = base0 + _splat(v * _LANES)
        bstar = jnp.where(found, base_v + pcnt - 1, bstar)
        pincl = jnp.where(found, pv + csel, pincl)
        hsel = jnp.where(found, hcur, hsel)
        pv = pv + _gat(cs, _splat(_LANES - 1))
        return pv, bstar, pincl, hsel

    init = (_splat(0), _splat(-1), _splat(0), _splat(0))
    return lax.fori_loop(0, 8192 // _LANES, body, init)


def _zero_hist(hist):
    zeros = jnp.zeros((_LANES,), jnp.int32)

    def zbody(i, _):
        for u in range(8):
            hist[pl.ds((i * 8 + u) * _LANES, _LANES)] = zeros
        return 0

    lax.fori_loop(0, _NBINS // _LANES // 8, zbody, 0)


_SH_ROUND = 16384


def _merge_scan(shared, sbuf, hist, s, s0, m_v):
    pv = _splat(0)
    bstar = _splat(0)
    pincl = _splat(0)
    hsel = _splat(0)
    for r in range(_NBINS // _SH_ROUND):
        pltpu.sync_copy(hist.at[pl.ds(r * _SH_ROUND, _SH_ROUND)],
                        shared.at[s])
        plsc.subcore_barrier()
        for j in range(_SH_ROUND // 8192):
            pltpu.sync_copy(shared.at[s0, pl.ds(j * 8192, 8192)],
                            sbuf.at[pl.ds(0, 8192)])
            pltpu.sync_copy(shared.at[s0 + 1, pl.ds(j * 8192, 8192)],
                            sbuf.at[pl.ds(8192, 8192)])
            base = r * _SH_ROUND + j * 8192
            cpv, cb, cp, ch = _scan_hist_pair(sbuf, m_v - pv, _splat(base))
            found = cb >= 0
            bstar = jnp.where(found, cb, bstar)
            pincl = jnp.where(found, pv + cp, pincl)
            hsel = jnp.where(found, ch, hsel)
            pv = pv + cpv
        plsc.subcore_barrier()
    return bstar, pincl, hsel


def _sc_select_body(x_hbm, res_hbm, buf, sbuf, hist, resv, shared,
                    *, k_rank, n_head):
    c = lax.axis_index("c")
    s = lax.axis_index("s")
    head = c * 8 + s // 2
    rowhalf = s % 2
    ncols = x_hbm.shape[2]
    rows_half = x_hbm.shape[1] // 2
    crows = _CHUNK // ncols
    row0 = rowhalf * rows_half
    nchunks = rows_half // crows
    ones = jnp.ones((_LANES,), jnp.int32)
    m1 = n_head - k_rank

    def hist_pass(mask_fn):
        def chunk_body(k, _):
            pltpu.sync_copy(
                x_hbm.at[head, pl.ds(row0 + k * crows, crows), :], buf)

            for r in range(crows):
                @plsc.parallel_loop(0, ncols // _LANES, step=1, unroll=8)
                def _(v):
                    xv = buf[r, pl.ds(v * _LANES, _LANES)]
                    uu = _sortable_i32(xv)
                    mask_fn(uu)

            return 0

        lax.fori_loop(0, nchunks, chunk_body, 0)

    _zero_hist(hist)

    def add_top16(uu):
        idx = _vshr_l(uu, 16)
        plsc.addupdate_scatter(hist, [idx], ones)

    hist_pass(add_top16)

    s0 = s - rowhalf
    m1_v = _splat(m1)
    bstar, pincl, hsel = _merge_scan(shared, sbuf, hist, s, s0, m1_v)

    _zero_hist(hist)

    def add_low16(uu):
        top = _vshr_l(uu, 16)
        idx = uu & jnp.int32(0xFFFF)
        plsc.addupdate_scatter(hist, [idx], ones, mask=top == bstar)

    hist_pass(add_low16)

    m2_v = hsel - (pincl - m1_v)
    bstar2, pincl2, hsel2 = _merge_scan(shared, sbuf, hist, s, s0, m2_v)

    u_v = jnp.left_shift(bstar, 16) | bstar2
    count_v = (_splat(n_head) - pincl) + hsel - (pincl2 - hsel2)

    iot = lax.iota(jnp.int32, _LANES)
    resv[...] = jnp.where(iot == 0, u_v, count_v)
    w = c * 16 + s
    pltpu.sync_copy(resv, res_hbm.at[w])


def _mask_kernel(thr_ref, x_ref, mask_ref):
    h = pl.program_id(0)
    mask_ref[0] = x_ref[0] >= thr_ref[h]


def kernel(batch_size, num_heads, seq_len, attention_scores):
    B, H, Sq, Skv = attention_scores.shape
    n = Sq * Skv
    BH = B * H
    x = attention_scores.reshape(BH, Sq, Skv)

    loc = np.float32(_THRESHOLD_PERCENTILE) * np.float32(n - 1)
    idx_hi = int(np.ceil(np.float64(loc)))
    k_rank = max(1, n - idx_hi)

    mesh = plsc.VectorSubcoreMesh(core_axis_name="c", subcore_axis_name="s")
    sc_select = functools.partial(
        pl.kernel,
        out_type=jax.ShapeDtypeStruct((32, _LANES), jnp.int32),
        mesh=mesh,
        compiler_params=pltpu.CompilerParams(needs_layout_passes=False),
        scratch_types=[
            pltpu.VMEM((_CHUNK // Skv, Skv), jnp.float32),
            pltpu.VMEM((2 * 8192,), jnp.int32),
            pltpu.VMEM((_NBINS,), jnp.int32),
            pltpu.VMEM((_LANES,), jnp.int32),
            pltpu.VMEM_SHARED((16, _SH_ROUND), jnp.int32),
        ],
    )(functools.partial(
        _sc_select_body, k_rank=k_rank, n_head=n))

    res = sc_select(x)

    rows = (jnp.arange(BH, dtype=jnp.int32) // 8) * 16 + (
        jnp.arange(BH, dtype=jnp.int32) % 8) * 2
    u_sort = res[rows, 0]
    counts = res[rows, 1]
    bits = jnp.where(
        u_sort < 0, u_sort ^ jnp.int32(-2147483648), ~u_sort
    )
    thr = lax.bitcast_convert_type(bits, jnp.float32)

    row_blk = 256
    mask3 = pl.pallas_call(
        _mask_kernel,
        grid=(BH, Sq // row_blk),
        in_specs=[
            pl.BlockSpec((BH,), lambda i, j: (0,), memory_space=pltpu.SMEM),
            pl.BlockSpec((1, row_blk, Skv), lambda i, j: (i, j, 0)),
        ],
        out_specs=pl.BlockSpec((1, row_blk, Skv), lambda i, j: (i, j, 0)),
        out_shape=jax.ShapeDtypeStruct((BH, Sq, Skv), jnp.bool_),
        compiler_params=pltpu.CompilerParams(
            dimension_semantics=("arbitrary", "arbitrary"),
            vmem_limit_bytes=60 * 1024 * 1024,
        ),
    )(thr, x)

    mask = mask3.reshape(B, H, Sq, Skv)
    density = jnp.sum(counts).astype(jnp.float32) / np.float32(BH * n)

    k = max(1, int(Skv * (1.0 - _SPARSITY_RATIO)))

    def topk_branch():
        _, topk_idx = jax.lax.top_k(attention_scores, k)
        bidx = jnp.arange(B)[:, None, None, None]
        hidx = jnp.arange(H)[None, :, None, None]
        qidx = jnp.arange(Sq)[None, None, :, None]
        topk_mask = jnp.zeros((B, H, Sq, Skv), dtype=bool)
        return topk_mask.at[bidx, hidx, qidx, topk_idx].set(True)

    return jax.lax.cond(
        density > np.float32(1.0 - _SPARSITY_RATIO),
        topk_branch,
        lambda: mask,
    )


# speedup vs baseline: 4.4686x; 1.4166x over previous
import functools

import jax
import jax.numpy as jnp
import numpy as np
from jax import lax
from jax.experimental import pallas as pl
from jax.experimental.pallas import tpu as pltpu
from jax.experimental.pallas import tpu_sc as plsc

_SPARSITY_RATIO = 0.9
_THRESHOLD_PERCENTILE = 0.95

_LANES = 16
_CHUNK = 16384
_NBINS = 65536


def _gat(x, lane_v):
    return lax.gather(
        x,
        lane_v[:, None],
        lax.GatherDimensionNumbers(
            offset_dims=(), collapsed_slice_dims=(0,), start_index_map=(0,)
        ),
        (1,),
        mode=lax.GatherScatterMode.PROMISE_IN_BOUNDS,
    )


def _splat(v):
    return jnp.full((_LANES,), v, jnp.int32)


def _vshr_l(x, k):
    return lax.shift_right_logical(x, jnp.full((_LANES,), k, jnp.int32))


def _sortable_i32(xv):
    b = lax.bitcast_convert_type(xv, jnp.int32)
    neg = lax.shift_right_arithmetic(b, jnp.full((_LANES,), 31, jnp.int32))
    return b ^ (neg | jnp.int32(-2147483648))


def _scan_hist_pair(sbuf, m_v, base0):
    def body(v, carry):
        pv, bstar, pincl, hsel = carry
        ha = sbuf[pl.ds(v * _LANES, _LANES)]
        hb = sbuf[pl.ds(8192 + v * _LANES, _LANES)]
        h = ha + hb
        cs = plsc.cumsum(h)
        pex = pv + cs - h
        cond = pex <= m_v
        pcnt = plsc.all_reduce_population_count(cond)
        found = pcnt > 0
        lane = jnp.maximum(pcnt - 1, 0)
        csel = _gat(cs, lane)
        hcur = _gat(h, lane)
        base_v = base0 + _splat(v * _LANES)
        bstar = jnp.where(found, base_v + pcnt - 1, bstar)
        pincl = jnp.where(found, pv + csel, pincl)
        hsel = jnp.where(found, hcur, hsel)
        pv = pv + _gat(cs, _splat(_LANES - 1))
        return pv, bstar, pincl, hsel

    init = (_splat(0), _splat(-1), _splat(0), _splat(0))
    return lax.fori_loop(0, 8192 // _LANES, body, init)


def _zero_hist(hist):
    zeros = jnp.zeros((_LANES,), jnp.int32)

    def zbody(i, _):
        for u in range(8):
            hist[pl.ds((i * 8 + u) * _LANES, _LANES)] = zeros
        return 0

    lax.fori_loop(0, _NBINS // _LANES // 8, zbody, 0)


_SH_ROUND = 8192


def _merge_scan(shared, sbuf, hist, s, s0, m_v):
    pv = _splat(0)
    bstar = _splat(0)
    pincl = _splat(0)
    hsel = _splat(0)
    for r in range(_NBINS // _SH_ROUND):
        pltpu.sync_copy(hist.at[pl.ds(r * _SH_ROUND, _SH_ROUND)],
                        shared.at[s])
        plsc.subcore_barrier()
        for j in range(_SH_ROUND // 8192):
            pltpu.sync_copy(shared.at[s0, pl.ds(j * 8192, 8192)],
                            sbuf.at[pl.ds(0, 8192)])
            pltpu.sync_copy(shared.at[s0 + 1, pl.ds(j * 8192, 8192)],
                            sbuf.at[pl.ds(8192, 8192)])
            base = r * _SH_ROUND + j * 8192
            cpv, cb, cp, ch = _scan_hist_pair(sbuf, m_v - pv, _splat(base))
            found = cb >= 0
            bstar = jnp.where(found, cb, bstar)
            pincl = jnp.where(found, pv + cp, pincl)
            hsel = jnp.where(found, ch, hsel)
            pv = pv + cpv
        plsc.subcore_barrier()
    return bstar, pincl, hsel


def _sc_select_body(x_hbm, res_hbm, buf0, buf1, sem0, sem1, sbuf, hist,
                    resv, shared, *, k_rank, n_head):
    c = lax.axis_index("c")
    s = lax.axis_index("s")
    head = c * 8 + s // 2
    rowhalf = s % 2
    ncols = x_hbm.shape[2]
    rows_half = x_hbm.shape[1] // 2
    crows = _CHUNK // ncols
    row0 = rowhalf * rows_half
    nchunks = rows_half // crows
    ones = jnp.ones((_LANES,), jnp.int32)
    m1 = n_head - k_rank

    bufs = (buf0, buf1)
    sems = (sem0, sem1)

    def hist_pass(mask_fn):
        def start(k, b):
            pltpu.make_async_copy(
                x_hbm.at[head, pl.ds(row0 + k * crows, crows), :],
                bufs[b], sems[b]).start()

        start(0, 0)
        start(1, 1)

        def chunk_body(k2, _):
            for b in range(2):
                k = k2 * 2 + b
                pltpu.make_async_copy(
                    x_hbm.at[head, pl.ds(row0, crows), :],
                    bufs[b], sems[b]).wait()

                for r in range(crows):
                    @plsc.parallel_loop(0, ncols // _LANES, step=1,
                                        unroll=8)
                    def _(v):
                        xv = bufs[b][r, pl.ds(v * _LANES, _LANES)]
                        uu = _sortable_i32(xv)
                        mask_fn(uu)

                @pl.when(k + 2 < nchunks)
                def _():
                    start(k + 2, b)

            return 0

        lax.fori_loop(0, nchunks // 2, chunk_body, 0)

    _zero_hist(hist)

    def add_top16(uu):
        idx = _vshr_l(uu, 16)
        plsc.addupdate_scatter(hist, [idx], ones)

    hist_pass(add_top16)

    s0 = s - rowhalf
    m1_v = _splat(m1)
    bstar, pincl, hsel = _merge_scan(shared, sbuf, hist, s, s0, m1_v)

    _zero_hist(hist)

    def add_low16(uu):
        top = _vshr_l(uu, 16)
        idx = uu & jnp.int32(0xFFFF)
        plsc.addupdate_scatter(hist, [idx], ones, mask=top == bstar)

    hist_pass(add_low16)

    m2_v = hsel - (pincl - m1_v)
    bstar2, pincl2, hsel2 = _merge_scan(shared, sbuf, hist, s, s0, m2_v)

    u_v = jnp.left_shift(bstar, 16) | bstar2
    count_v = (_splat(n_head) - pincl) + hsel - (pincl2 - hsel2)

    iot = lax.iota(jnp.int32, _LANES)
    resv[...] = jnp.where(iot == 0, u_v, count_v)
    w = c * 16 + s
    pltpu.sync_copy(resv, res_hbm.at[w])


def _mask_kernel(thr_ref, x_ref, mask_ref):
    h = pl.program_id(0)
    mask_ref[0] = x_ref[0] >= thr_ref[h]


def kernel(batch_size, num_heads, seq_len, attention_scores):
    B, H, Sq, Skv = attention_scores.shape
    n = Sq * Skv
    BH = B * H
    x = attention_scores.reshape(BH, Sq, Skv)

    loc = np.float32(_THRESHOLD_PERCENTILE) * np.float32(n - 1)
    idx_hi = int(np.ceil(np.float64(loc)))
    k_rank = max(1, n - idx_hi)

    mesh = plsc.VectorSubcoreMesh(core_axis_name="c", subcore_axis_name="s")
    sc_select = functools.partial(
        pl.kernel,
        out_type=jax.ShapeDtypeStruct((32, _LANES), jnp.int32),
        mesh=mesh,
        compiler_params=pltpu.CompilerParams(needs_layout_passes=False),
        scratch_types=[
            pltpu.VMEM((_CHUNK // Skv, Skv), jnp.float32),
            pltpu.VMEM((_CHUNK // Skv, Skv), jnp.float32),
            pltpu.SemaphoreType.DMA,
            pltpu.SemaphoreType.DMA,
            pltpu.VMEM((2 * 8192,), jnp.int32),
            pltpu.VMEM((_NBINS,), jnp.int32),
            pltpu.VMEM((_LANES,), jnp.int32),
            pltpu.VMEM_SHARED((16, _SH_ROUND), jnp.int32),
        ],
    )(functools.partial(
        _sc_select_body, k_rank=k_rank, n_head=n))

    res = sc_select(x)

    rows = (jnp.arange(BH, dtype=jnp.int32) // 8) * 16 + (
        jnp.arange(BH, dtype=jnp.int32) % 8) * 2
    u_sort = res[rows, 0]
    counts = res[rows, 1]
    bits = jnp.where(
        u_sort < 0, u_sort ^ jnp.int32(-2147483648), ~u_sort
    )
    thr = lax.bitcast_convert_type(bits, jnp.float32)

    row_blk = 256
    mask3 = pl.pallas_call(
        _mask_kernel,
        grid=(BH, Sq // row_blk),
        in_specs=[
            pl.BlockSpec((BH,), lambda i, j: (0,), memory_space=pltpu.SMEM),
            pl.BlockSpec((1, row_blk, Skv), lambda i, j: (i, j, 0)),
        ],
        out_specs=pl.BlockSpec((1, row_blk, Skv), lambda i, j: (i, j, 0)),
        out_shape=jax.ShapeDtypeStruct((BH, Sq, Skv), jnp.bool_),
        compiler_params=pltpu.CompilerParams(
            dimension_semantics=("arbitrary", "arbitrary"),
            vmem_limit_bytes=60 * 1024 * 1024,
        ),
    )(thr, x)

    mask = mask3.reshape(B, H, Sq, Skv)
    density = jnp.sum(counts).astype(jnp.float32) / np.float32(BH * n)

    k = max(1, int(Skv * (1.0 - _SPARSITY_RATIO)))

    def topk_branch():
        _, topk_idx = jax.lax.top_k(attention_scores, k)
        bidx = jnp.arange(B)[:, None, None, None]
        hidx = jnp.arange(H)[None, :, None, None]
        qidx = jnp.arange(Sq)[None, None, :, None]
        topk_mask = jnp.zeros((B, H, Sq, Skv), dtype=bool)
        return topk_mask.at[bidx, hidx, qidx, topk_idx].set(True)

    return jax.lax.cond(
        density > np.float32(1.0 - _SPARSITY_RATIO),
        topk_branch,
        lambda: mask,
    )


# speedup vs baseline: 4.5716x; 1.0230x over previous
import functools

import jax
import jax.numpy as jnp
import numpy as np
from jax import lax
from jax.experimental import pallas as pl
from jax.experimental.pallas import tpu as pltpu
from jax.experimental.pallas import tpu_sc as plsc

_SPARSITY_RATIO = 0.9
_THRESHOLD_PERCENTILE = 0.95

_LANES = 16
_CHUNK = 16384
_NBINS = 65536


def _gat(x, lane_v):
    return lax.gather(
        x,
        lane_v[:, None],
        lax.GatherDimensionNumbers(
            offset_dims=(), collapsed_slice_dims=(0,), start_index_map=(0,)
        ),
        (1,),
        mode=lax.GatherScatterMode.PROMISE_IN_BOUNDS,
    )


def _splat(v):
    return jnp.full((_LANES,), v, jnp.int32)


def _vshr_l(x, k):
    return lax.shift_right_logical(x, jnp.full((_LANES,), k, jnp.int32))


def _sortable_i32(xv):
    b = lax.bitcast_convert_type(xv, jnp.int32)
    neg = lax.shift_right_arithmetic(b, jnp.full((_LANES,), 31, jnp.int32))
    return b ^ (neg | jnp.int32(-2147483648))


def _scan_hist_pair(sbuf, m_v, base0):
    def body(v, carry):
        pv, bstar, pincl, hsel = carry
        ha = sbuf[pl.ds(v * _LANES, _LANES)]
        hb = sbuf[pl.ds(8192 + v * _LANES, _LANES)]
        h = ha + hb
        cs = plsc.cumsum(h)
        pex = pv + cs - h
        cond = pex <= m_v
        pcnt = plsc.all_reduce_population_count(cond)
        found = pcnt > 0
        lane = jnp.maximum(pcnt - 1, 0)
        csel = _gat(cs, lane)
        hcur = _gat(h, lane)
        base_v = base0 + _splat(v * _LANES)
        bstar = jnp.where(found, base_v + pcnt - 1, bstar)
        pincl = jnp.where(found, pv + csel, pincl)
        hsel = jnp.where(found, hcur, hsel)
        pv = pv + _gat(cs, _splat(_LANES - 1))
        return pv, bstar, pincl, hsel

    init = (_splat(0), _splat(-1), _splat(0), _splat(0))
    return lax.fori_loop(0, 8192 // _LANES, body, init)


def _zero_hist(hist):
    zeros = jnp.zeros((_LANES,), jnp.int32)

    def zbody(i, _):
        for u in range(8):
            hist[pl.ds((i * 8 + u) * _LANES, _LANES)] = zeros
        return 0

    lax.fori_loop(0, _NBINS // _LANES // 8, zbody, 0)


_SH_ROUND = 8192


def _merge_scan(shared, sbuf, hist, s, s0, m_v):
    pv = _splat(0)
    bstar = _splat(0)
    pincl = _splat(0)
    hsel = _splat(0)
    for r in range(_NBINS // _SH_ROUND):
        pltpu.sync_copy(hist.at[pl.ds(r * _SH_ROUND, _SH_ROUND)],
                        shared.at[s])
        plsc.subcore_barrier()
        for j in range(_SH_ROUND // 8192):
            pltpu.sync_copy(shared.at[s0, pl.ds(j * 8192, 8192)],
                            sbuf.at[pl.ds(0, 8192)])
            pltpu.sync_copy(shared.at[s0 + 1, pl.ds(j * 8192, 8192)],
                            sbuf.at[pl.ds(8192, 8192)])
            base = r * _SH_ROUND + j * 8192
            cpv, cb, cp, ch = _scan_hist_pair(sbuf, m_v - pv, _splat(base))
            found = cb >= 0
            bstar = jnp.where(found, cb, bstar)
            pincl = jnp.where(found, pv + cp, pincl)
            hsel = jnp.where(found, ch, hsel)
            pv = pv + cpv
        plsc.subcore_barrier()
    return bstar, pincl, hsel


def _sc_select_body(x_hbm, res_hbm, buf0, buf1, sem0, sem1, sbuf, hist,
                    resv, shared, *, k_rank, n_head):
    c = lax.axis_index("c")
    s = lax.axis_index("s")
    head = c * 8 + s // 2
    rowhalf = s % 2
    ncols = x_hbm.shape[2]
    rows_half = x_hbm.shape[1] // 2
    crows = _CHUNK // ncols
    row0 = rowhalf * rows_half
    nchunks = rows_half // crows
    ones = jnp.ones((_LANES,), jnp.int32)
    m1 = n_head - k_rank

    bufs = (buf0, buf1)
    sems = (sem0, sem1)

    def hist_pass(mask_fn):
        def start(k, b):
            pltpu.make_async_copy(
                x_hbm.at[head, pl.ds(row0 + k * crows, crows), :],
                bufs[b], sems[b]).start()

        start(0, 0)
        start(1, 1)

        def chunk_body(k2, _):
            for b in range(2):
                k = k2 * 2 + b
                pltpu.make_async_copy(
                    x_hbm.at[head, pl.ds(row0, crows), :],
                    bufs[b], sems[b]).wait()

                vpr = ncols // _LANES

                @plsc.parallel_loop(0, crows * vpr, step=1, unroll=8)
                def _(v):
                    r = v // vpr
                    col = v % vpr
                    xv = bufs[b][r, pl.ds(col * _LANES, _LANES)]
                    uu = _sortable_i32(xv)
                    mask_fn(uu)

                @pl.when(k + 2 < nchunks)
                def _():
                    start(k + 2, b)

            return 0

        lax.fori_loop(0, nchunks // 2, chunk_body, 0)

    _zero_hist(hist)

    def add_top16(uu):
        idx = _vshr_l(uu, 16)
        plsc.addupdate_scatter(hist, [idx], ones)

    hist_pass(add_top16)

    s0 = s - rowhalf
    m1_v = _splat(m1)
    bstar, pincl, hsel = _merge_scan(shared, sbuf, hist, s, s0, m1_v)

    _zero_hist(hist)

    def add_low16(uu):
        top = _vshr_l(uu, 16)
        idx = uu & jnp.int32(0xFFFF)
        plsc.addupdate_scatter(hist, [idx], ones, mask=top == bstar)

    hist_pass(add_low16)

    m2_v = hsel - (pincl - m1_v)
    bstar2, pincl2, hsel2 = _merge_scan(shared, sbuf, hist, s, s0, m2_v)

    u_v = jnp.left_shift(bstar, 16) | bstar2
    count_v = (_splat(n_head) - pincl) + hsel - (pincl2 - hsel2)

    iot = lax.iota(jnp.int32, _LANES)
    resv[...] = jnp.where(iot == 0, u_v, count_v)
    w = c * 16 + s
    pltpu.sync_copy(resv, res_hbm.at[w])


def _mask_kernel(thr_ref, x_ref, mask_ref):
    h = pl.program_id(0)
    mask_ref[0] = x_ref[0] >= thr_ref[h]


def kernel(batch_size, num_heads, seq_len, attention_scores):
    B, H, Sq, Skv = attention_scores.shape
    n = Sq * Skv
    BH = B * H
    x = attention_scores.reshape(BH, Sq, Skv)

    loc = np.float32(_THRESHOLD_PERCENTILE) * np.float32(n - 1)
    idx_hi = int(np.ceil(np.float64(loc)))
    k_rank = max(1, n - idx_hi)

    mesh = plsc.VectorSubcoreMesh(core_axis_name="c", subcore_axis_name="s")
    sc_select = functools.partial(
        pl.kernel,
        out_type=jax.ShapeDtypeStruct((32, _LANES), jnp.int32),
        mesh=mesh,
        compiler_params=pltpu.CompilerParams(needs_layout_passes=False),
        scratch_types=[
            pltpu.VMEM((_CHUNK // Skv, Skv), jnp.float32),
            pltpu.VMEM((_CHUNK // Skv, Skv), jnp.float32),
            pltpu.SemaphoreType.DMA,
            pltpu.SemaphoreType.DMA,
            pltpu.VMEM((2 * 8192,), jnp.int32),
            pltpu.VMEM((_NBINS,), jnp.int32),
            pltpu.VMEM((_LANES,), jnp.int32),
            pltpu.VMEM_SHARED((16, _SH_ROUND), jnp.int32),
        ],
    )(functools.partial(
        _sc_select_body, k_rank=k_rank, n_head=n))

    res = sc_select(x)

    rows = (jnp.arange(BH, dtype=jnp.int32) // 8) * 16 + (
        jnp.arange(BH, dtype=jnp.int32) % 8) * 2
    u_sort = res[rows, 0]
    counts = res[rows, 1]
    bits = jnp.where(
        u_sort < 0, u_sort ^ jnp.int32(-2147483648), ~u_sort
    )
    thr = lax.bitcast_convert_type(bits, jnp.float32)

    row_blk = 256
    mask3 = pl.pallas_call(
        _mask_kernel,
        grid=(BH, Sq // row_blk),
        in_specs=[
            pl.BlockSpec((BH,), lambda i, j: (0,), memory_space=pltpu.SMEM),
            pl.BlockSpec((1, row_blk, Skv), lambda i, j: (i, j, 0)),
        ],
        out_specs=pl.BlockSpec((1, row_blk, Skv), lambda i, j: (i, j, 0)),
        out_shape=jax.ShapeDtypeStruct((BH, Sq, Skv), jnp.bool_),
        compiler_params=pltpu.CompilerParams(
            dimension_semantics=("arbitrary", "arbitrary"),
            vmem_limit_bytes=60 * 1024 * 1024,
        ),
    )(thr, x)

    mask = mask3.reshape(B, H, Sq, Skv)
    density = jnp.sum(counts).astype(jnp.float32) / np.float32(BH * n)

    k = max(1, int(Skv * (1.0 - _SPARSITY_RATIO)))

    def topk_branch():
        _, topk_idx = jax.lax.top_k(attention_scores, k)
        bidx = jnp.arange(B)[:, None, None, None]
        hidx = jnp.arange(H)[None, :, None, None]
        qidx = jnp.arange(Sq)[None, None, :, None]
        topk_mask = jnp.zeros((B, H, Sq, Skv), dtype=bool)
        return topk_mask.at[bidx, hidx, qidx, topk_idx].set(True)

    return jax.lax.cond(
        density > np.float32(1.0 - _SPARSITY_RATIO),
        topk_branch,
        lambda: mask,
    )


# speedup vs baseline: 4.6596x; 1.0193x over previous
import functools

import jax
import jax.numpy as jnp
import numpy as np
from jax import lax
from jax.experimental import pallas as pl
from jax.experimental.pallas import tpu as pltpu
from jax.experimental.pallas import tpu_sc as plsc

_SPARSITY_RATIO = 0.9
_THRESHOLD_PERCENTILE = 0.95

_LANES = 16
_CHUNK = 16384
_NBINS = 65536


def _gat(x, lane_v):
    return lax.gather(
        x,
        lane_v[:, None],
        lax.GatherDimensionNumbers(
            offset_dims=(), collapsed_slice_dims=(0,), start_index_map=(0,)
        ),
        (1,),
        mode=lax.GatherScatterMode.PROMISE_IN_BOUNDS,
    )


def _splat(v):
    return jnp.full((_LANES,), v, jnp.int32)


def _vshr_l(x, k):
    return lax.shift_right_logical(x, jnp.full((_LANES,), k, jnp.int32))


def _sortable_i32(xv):
    b = lax.bitcast_convert_type(xv, jnp.int32)
    neg = lax.shift_right_arithmetic(b, jnp.full((_LANES,), 31, jnp.int32))
    return b ^ (neg | jnp.int32(-2147483648))


def _scan_hist_pair(sbuf, tbuf, m_s, base0):
    nv = 8192 // _LANES

    @plsc.parallel_loop(0, nv, step=1, unroll=4)
    def _(v):
        h = sbuf[pl.ds(v * _LANES, _LANES)] + \
            sbuf[pl.ds(8192 + v * _LANES, _LANES)]
        tbuf[v] = jnp.sum(h)

    def body(i, carry):
        pv, fvec, fpre = carry
        t = tbuf[i]
        found = pv <= m_s
        fvec = jnp.where(found, i, fvec)
        fpre = jnp.where(found, pv, fpre)
        return pv + t, fvec, fpre

    pv, fvec, fpre = lax.fori_loop(
        0, nv, body, (jnp.int32(0), jnp.int32(-1), jnp.int32(0)),
        unroll=8)

    voff = jnp.maximum(fvec, 0) * _LANES
    h = sbuf[pl.ds(voff, _LANES)] + sbuf[pl.ds(8192 + voff, _LANES)]
    cs = plsc.cumsum(h)
    pex = _splat(fpre) + cs - h
    cond = pex <= _splat(m_s)
    pcnt = plsc.all_reduce_population_count(cond)
    lane = jnp.maximum(pcnt - 1, 0)
    found = fvec >= 0
    bstar = jnp.where(found,
                      base0 + fvec * _LANES + jnp.max(pcnt) - 1,
                      jnp.int32(-1))
    pincl = jnp.where(found, fpre + jnp.max(_gat(cs, lane)), jnp.int32(0))
    hsel = jnp.where(found, jnp.max(_gat(h, lane)), jnp.int32(0))
    return pv, bstar, pincl, hsel


def _zero_hist(hist):
    zeros = jnp.zeros((_LANES,), jnp.int32)

    def zbody(i, _):
        for u in range(8):
            hist[pl.ds((i * 8 + u) * _LANES, _LANES)] = zeros
        return 0

    lax.fori_loop(0, _NBINS // _LANES // 8, zbody, 0)


_SH_ROUND = 8192


def _merge_scan(shared, sbuf, tbuf, hist, s, s0, m_s):
    pv = jnp.int32(0)
    bstar = jnp.int32(0)
    pincl = jnp.int32(0)
    hsel = jnp.int32(0)
    for r in range(_NBINS // _SH_ROUND):
        pltpu.sync_copy(hist.at[pl.ds(r * _SH_ROUND, _SH_ROUND)],
                        shared.at[s])
        plsc.subcore_barrier()
        for j in range(_SH_ROUND // 8192):
            pltpu.sync_copy(shared.at[s0, pl.ds(j * 8192, 8192)],
                            sbuf.at[pl.ds(0, 8192)])
            pltpu.sync_copy(shared.at[s0 + 1, pl.ds(j * 8192, 8192)],
                            sbuf.at[pl.ds(8192, 8192)])
            base = r * _SH_ROUND + j * 8192
            cpv, cb, cp, ch = _scan_hist_pair(sbuf, tbuf, m_s - pv, base)
            found = cb >= 0
            bstar = jnp.where(found, cb, bstar)
            pincl = jnp.where(found, pv + cp, pincl)
            hsel = jnp.where(found, ch, hsel)
            pv = pv + cpv
        plsc.subcore_barrier()
    return bstar, pincl, hsel


def _sc_select_body(x_hbm, res_hbm, buf0, buf1, sem0, sem1, sbuf, tbuf,
                    hist, resv, shared, *, k_rank, n_head):
    c = lax.axis_index("c")
    s = lax.axis_index("s")
    head = c * 8 + s // 2
    rowhalf = s % 2
    ncols = x_hbm.shape[2]
    rows_half = x_hbm.shape[1] // 2
    crows = _CHUNK // ncols
    row0 = rowhalf * rows_half
    nchunks = rows_half // crows
    ones = jnp.ones((_LANES,), jnp.int32)
    m1 = n_head - k_rank

    bufs = (buf0, buf1)
    sems = (sem0, sem1)

    def hist_pass(mask_fn):
        def start(k, b):
            pltpu.make_async_copy(
                x_hbm.at[head, pl.ds(row0 + k * crows, crows), :],
                bufs[b], sems[b]).start()

        start(0, 0)
        start(1, 1)

        def chunk_body(k2, _):
            for b in range(2):
                k = k2 * 2 + b
                pltpu.make_async_copy(
                    x_hbm.at[head, pl.ds(row0, crows), :],
                    bufs[b], sems[b]).wait()

                vpr = ncols // _LANES

                @plsc.parallel_loop(0, crows * vpr, step=1, unroll=8)
                def _(v):
                    r = v // vpr
                    col = v % vpr
                    xv = bufs[b][r, pl.ds(col * _LANES, _LANES)]
                    uu = _sortable_i32(xv)
                    mask_fn(uu)

                @pl.when(k + 2 < nchunks)
                def _():
                    start(k + 2, b)

            return 0

        lax.fori_loop(0, nchunks // 2, chunk_body, 0)

    _zero_hist(hist)

    def add_top16(uu):
        idx = _vshr_l(uu, 16)
        plsc.addupdate_scatter(hist, [idx], ones)

    hist_pass(add_top16)

    s0 = s - rowhalf
    bstar, pincl, hsel = _merge_scan(shared, sbuf, tbuf, hist, s, s0,
                                     jnp.int32(m1))

    _zero_hist(hist)
    bstar_v = _splat(bstar)

    def add_low16(uu):
        top = _vshr_l(uu, 16)
        idx = uu & jnp.int32(0xFFFF)
        plsc.addupdate_scatter(hist, [idx], ones, mask=top == bstar_v)

    hist_pass(add_low16)

    m2 = hsel - (pincl - jnp.int32(m1))
    bstar2, pincl2, hsel2 = _merge_scan(shared, sbuf, tbuf, hist, s, s0,
                                        m2)

    u_s = jnp.left_shift(bstar, 16) | bstar2
    count_s = (jnp.int32(n_head) - pincl) + hsel - (pincl2 - hsel2)

    iot = lax.iota(jnp.int32, _LANES)
    resv[...] = jnp.where(iot == 0, u_s, count_s)
    w = c * 16 + s
    pltpu.sync_copy(resv, res_hbm.at[w])


def _mask_kernel(thr_ref, x_ref, mask_ref):
    h = pl.program_id(0)
    mask_ref[0] = x_ref[0] >= thr_ref[h]


def kernel(batch_size, num_heads, seq_len, attention_scores):
    B, H, Sq, Skv = attention_scores.shape
    n = Sq * Skv
    BH = B * H
    x = attention_scores.reshape(BH, Sq, Skv)

    loc = np.float32(_THRESHOLD_PERCENTILE) * np.float32(n - 1)
    idx_hi = int(np.ceil(np.float64(loc)))
    k_rank = max(1, n - idx_hi)

    mesh = plsc.VectorSubcoreMesh(core_axis_name="c", subcore_axis_name="s")
    sc_select = functools.partial(
        pl.kernel,
        out_type=jax.ShapeDtypeStruct((32, _LANES), jnp.int32),
        mesh=mesh,
        compiler_params=pltpu.CompilerParams(needs_layout_passes=False),
        scratch_types=[
            pltpu.VMEM((_CHUNK // Skv, Skv), jnp.float32),
            pltpu.VMEM((_CHUNK // Skv, Skv), jnp.float32),
            pltpu.SemaphoreType.DMA,
            pltpu.SemaphoreType.DMA,
            pltpu.VMEM((2 * 8192,), jnp.int32),
            pltpu.SMEM((8192 // _LANES,), jnp.int32),
            pltpu.VMEM((_NBINS,), jnp.int32),
            pltpu.VMEM((_LANES,), jnp.int32),
            pltpu.VMEM_SHARED((16, _SH_ROUND), jnp.int32),
        ],
    )(functools.partial(
        _sc_select_body, k_rank=k_rank, n_head=n))

    res = sc_select(x)

    rows = (jnp.arange(BH, dtype=jnp.int32) // 8) * 16 + (
        jnp.arange(BH, dtype=jnp.int32) % 8) * 2
    u_sort = res[rows, 0]
    counts = res[rows, 1]
    bits = jnp.where(
        u_sort < 0, u_sort ^ jnp.int32(-2147483648), ~u_sort
    )
    thr = lax.bitcast_convert_type(bits, jnp.float32)

    row_blk = 256
    mask3 = pl.pallas_call(
        _mask_kernel,
        grid=(BH, Sq // row_blk),
        in_specs=[
            pl.BlockSpec((BH,), lambda i, j: (0,), memory_space=pltpu.SMEM),
            pl.BlockSpec((1, row_blk, Skv), lambda i, j: (i, j, 0)),
        ],
        out_specs=pl.BlockSpec((1, row_blk, Skv), lambda i, j: (i, j, 0)),
        out_shape=jax.ShapeDtypeStruct((BH, Sq, Skv), jnp.bool_),
        compiler_params=pltpu.CompilerParams(
            dimension_semantics=("arbitrary", "arbitrary"),
            vmem_limit_bytes=60 * 1024 * 1024,
        ),
    )(thr, x)

    mask = mask3.reshape(B, H, Sq, Skv)
    density = jnp.sum(counts).astype(jnp.float32) / np.float32(BH * n)

    k = max(1, int(Skv * (1.0 - _SPARSITY_RATIO)))

    def topk_branch():
        _, topk_idx = jax.lax.top_k(attention_scores, k)
        bidx = jnp.arange(B)[:, None, None, None]
        hidx = jnp.arange(H)[None, :, None, None]
        qidx = jnp.arange(Sq)[None, None, :, None]
        topk_mask = jnp.zeros((B, H, Sq, Skv), dtype=bool)
        return topk_mask.at[bidx, hidx, qidx, topk_idx].set(True)

    return jax.lax.cond(
        density > np.float32(1.0 - _SPARSITY_RATIO),
        topk_branch,
        lambda: mask,
    )
